# baseline (speedup 1.0000x reference)
import functools

import jax
import jax.numpy as jnp
from jax import lax
from jax.experimental import pallas as pl
from jax.experimental.pallas import tpu as pltpu
from jax.experimental.pallas import tpu_sc as plsc

H = 512
W = 512
C = 3
PH = 64
PW = 64
NB = 64
NS = 2
NK = NB * NS
PADH = H + PH - 1
PADW = W + PW - 1

NUM_WORKERS = 32
PATCHES_PER_WORKER = NK // NUM_WORKERS

_scalar_mesh = plsc.ScalarSubcoreMesh(axis_name="c", num_cores=2)
_vector_mesh = plsc.VectorSubcoreMesh(core_axis_name="c", subcore_axis_name="s")
_sc_params = pltpu.CompilerParams(
    use_tc_tiling_on_sc=False, needs_layout_passes=False)


@functools.partial(
    pl.kernel,
    out_type=jax.ShapeDtypeStruct((PADH, PADW, C), jnp.float32),
    mesh=_scalar_mesh,
    scratch_types=[pltpu.SemaphoreType.DMA],
    compiler_params=_sc_params,
)
def _pad_kernel(img_hbm, pad_hbm, sem):
    c = lax.axis_index("c")

    @pl.when(c == 0)
    def _():
        c1 = pltpu.async_copy(
            img_hbm, pad_hbm.at[pl.ds(0, H), pl.ds(0, W), :], sem)
        c2 = pltpu.async_copy(
            img_hbm.at[:, pl.ds(0, PW - 1), :],
            pad_hbm.at[pl.ds(0, H), pl.ds(W, PW - 1), :], sem)
        c1.wait()
        c2.wait()

    @pl.when(c == 1)
    def _():
        c3 = pltpu.async_copy(
            img_hbm.at[pl.ds(0, PH - 1), :, :],
            pad_hbm.at[pl.ds(H, PH - 1), pl.ds(0, W), :], sem)
        c4 = pltpu.async_copy(
            img_hbm.at[pl.ds(0, PH - 1), pl.ds(0, PW - 1), :],
            pad_hbm.at[pl.ds(H, PH - 1), pl.ds(W, PW - 1), :], sem)
        c3.wait()
        c4.wait()


@functools.partial(
    pl.kernel,
    out_type=jax.ShapeDtypeStruct((NB, PH, PW, NS * C), jnp.float32),
    mesh=_vector_mesh,
    scratch_types=[
        pltpu.VMEM((NK,), jnp.int32),
        pltpu.VMEM((NK,), jnp.int32),
        pltpu.VMEM((PH, PW, NS * C), jnp.float32),
        pltpu.SemaphoreType.DMA,
        pltpu.SemaphoreType.DMA,
    ],
    compiler_params=_sc_params,
)
def _gather_kernel(pad_hbm, ys_hbm, xs_hbm, out_hbm, ys_v, xs_v,
                   obuf, sem_in, sem_out):
    wid = lax.axis_index("s") * 2 + lax.axis_index("c")

    cy = pltpu.async_copy(ys_hbm, ys_v, sem_in)
    cx = pltpu.async_copy(xs_hbm, xs_v, sem_in)
    cy.wait()
    cx.wait()

    grp = 16 * (wid // 4)
    ys_grp = ys_v[pl.ds(grp, 16)]
    xs_grp = xs_v[pl.ds(grp, 16)]
    lanes = lax.iota(jnp.int32, 16)
    lane_base = (wid % 4) * 4

    for u in range(NB // NUM_WORKERS):
        b = wid * (NB // NUM_WORKERS) + u
        in_copies = []
        for s in range(NS):
            t = u * NS + s
            y = jnp.sum(jnp.where(lanes == lane_base + t, ys_grp, 0))
            x = jnp.sum(jnp.where(lanes == lane_base + t, xs_grp, 0))
            in_copies.append(pltpu.async_copy(
                pad_hbm.at[pl.ds(y, PH), pl.ds(x, PW), :],
                obuf.at[:, :, pl.ds(s * C, C)], sem_in))
        for s in range(NS):
            in_copies[s].wait()
        pltpu.async_copy(obuf, out_hbm.at[b], sem_out).wait()


def kernel(img, dummy, ys, xs):
    del dummy
    img3 = img.reshape(H, W, C)
    pad = _pad_kernel(img3)
    return _gather_kernel(pad, ys, xs)


# speedup vs baseline: 2.6641x; 2.6641x over previous
import functools

import jax
import jax.numpy as jnp
from jax import lax
from jax.experimental import pallas as pl
from jax.experimental.pallas import tpu as pltpu
from jax.experimental.pallas import tpu_sc as plsc

H = 512
W = 512
C = 3
PH = 64
PW = 64
NB = 64
NS = 2
NK = NB * NS
PADH = 576
PADW = W + PW - 1

NUM_WORKERS = 32
ROW_F32 = PW * C
OROW_F32 = PW * NS * C
BLK_F32 = PH * OROW_F32

_vector_mesh = plsc.VectorSubcoreMesh(core_axis_name="c", subcore_axis_name="s")
_sc_params = pltpu.CompilerParams(
    use_tc_tiling_on_sc=False, needs_layout_passes=False)


@functools.partial(
    pl.kernel,
    out_type=jax.ShapeDtypeStruct((PADH, PADW, C), jnp.float32),
    mesh=_vector_mesh,
    scratch_types=[
        pltpu.VMEM((16, PADW, C), jnp.float32),
        pltpu.VMEM((2, PADW, C), jnp.float32),
        pltpu.SemaphoreType.DMA,
        pltpu.SemaphoreType.DMA,
    ],
    compiler_params=_sc_params,
)
def _pad_kernel(img_hbm, pad_hbm, rbuf, wbuf, sem_in, sem_out):
    wid = lax.axis_index("s") * 2 + lax.axis_index("c")

    r0 = wid * 16
    c1 = pltpu.async_copy(
        img_hbm.at[pl.ds(r0, 16), :, :], rbuf.at[:, pl.ds(0, W), :], sem_in)
    c2 = pltpu.async_copy(
        img_hbm.at[pl.ds(r0, 16), pl.ds(0, PW - 1), :],
        rbuf.at[:, pl.ds(W, PW - 1), :], sem_in)
    r2 = wid * 2
    c3 = pltpu.async_copy(
        img_hbm.at[pl.ds(r2, 2), :, :], wbuf.at[:, pl.ds(0, W), :], sem_in)
    c4 = pltpu.async_copy(
        img_hbm.at[pl.ds(r2, 2), pl.ds(0, PW - 1), :],
        wbuf.at[:, pl.ds(W, PW - 1), :], sem_in)
    c1.wait()
    c2.wait()
    o1 = pltpu.async_copy(rbuf, pad_hbm.at[pl.ds(r0, 16)], sem_out)
    c3.wait()
    c4.wait()
    o2 = pltpu.async_copy(wbuf, pad_hbm.at[pl.ds(H + r2, 2)], sem_out)
    o1.wait()
    o2.wait()


@functools.partial(
    pl.kernel,
    out_type=jax.ShapeDtypeStruct((NB, BLK_F32), jnp.float32),
    mesh=_vector_mesh,
    scratch_types=[
        pltpu.VMEM((NK,), jnp.int32),
        pltpu.VMEM((NK,), jnp.int32),
        pltpu.VMEM((PH, PW, C), jnp.float32),
        pltpu.VMEM((PH, PW, C), jnp.float32),
        pltpu.VMEM((BLK_F32,), jnp.float32),
        pltpu.VMEM((BLK_F32,), jnp.float32),
        pltpu.SemaphoreType.DMA,
        pltpu.SemaphoreType.DMA,
    ],
    compiler_params=_sc_params,
)
def _gather_kernel(pad_hbm, ys_hbm, xs_hbm, out_hbm, ys_v, xs_v,
                   pb0, pb1, ob0, ob1, sem_in, sem_out):
    wid = lax.axis_index("s") * 2 + lax.axis_index("c")

    cy = pltpu.async_copy(ys_hbm, ys_v, sem_in)
    cx = pltpu.async_copy(xs_hbm, xs_v, sem_in)
    cy.wait()
    cx.wait()

    grp = 16 * (wid // 4)
    ys_grp = ys_v[pl.ds(grp, 16)]
    xs_grp = xs_v[pl.ds(grp, 16)]
    lane = lax.iota(jnp.int32, 16)
    lane_base = (wid % 4) * 4

    pbufs = (pb0, pb1)
    obufs = (ob0, ob1)

    def patch_dma(u, s):
        t = u * NS + s
        y = jnp.sum(jnp.where(lane == lane_base + t, ys_grp, 0))
        x = jnp.sum(jnp.where(lane == lane_base + t, xs_grp, 0))
        src = pad_hbm.at[pl.ds(y, PH), pl.ds(x, PW), :]
        return pltpu.async_copy(src, pbufs[s], sem_in)

    src_j = []
    src_c = []
    dst_base = []
    for v in range(ROW_F32 // 16):
        q = 16 * v + lane
        qm = lax.rem(q, 3)
        src_j.append(lax.div(q, 3))
        src_c.append(qm)
        dst_base.append(2 * q - qm)

    out_copies = []
    for u in range(2):
        b = wid * 2 + u
        in_copies = [patch_dma(u, s) for s in range(NS)]
        for s in range(NS):
            in_copies[s].wait()

        def body(i, u=u):
            i_vec = jnp.full((16,), i, dtype=jnp.int32)
            off = jnp.full((16,), i * OROW_F32, dtype=jnp.int32)
            for s in range(NS):
                for v in range(ROW_F32 // 16):
                    x = plsc.load_gather(
                        pbufs[s], [i_vec, src_j[v], src_c[v]])
                    plsc.store_scatter(
                        obufs[u], [dst_base[v] + off + 3 * s], x)

        pl.loop(0, PH)(body)
        out_copies.append(
            pltpu.async_copy(obufs[u], out_hbm.at[b], sem_out))
    for cpy in out_copies:
        cpy.wait()


def kernel(img, dummy, ys, xs):
    del dummy
    img3 = img.reshape(H, W, C)
    pad = _pad_kernel(img3)
    out = _gather_kernel(pad, ys, xs)
    return out.reshape(NB, PH, PW, NS * C)


# speedup vs baseline: 14.9667x; 5.6179x over previous
import functools

import jax
import jax.numpy as jnp
from jax import lax
from jax.experimental import pallas as pl
from jax.experimental.pallas import tpu as pltpu
from jax.experimental.pallas import tpu_sc as plsc

H = 512
W = 512
C = 3
PH = 64
PW = 64
NB = 64
NS = 2
NK = NB * NS
PADH = 576
PADW = 576
WIN = PW + 8

NUM_WORKERS = 32

_vector_mesh = plsc.VectorSubcoreMesh(core_axis_name="c", subcore_axis_name="s")
_sc_params = pltpu.CompilerParams(
    use_tc_tiling_on_sc=False, needs_layout_passes=False)


@functools.partial(
    pl.kernel,
    out_type=jax.ShapeDtypeStruct((C, PADH, PADW), jnp.float32),
    mesh=_vector_mesh,
    scratch_types=[
        pltpu.VMEM((C, 16, PADW), jnp.float32),
        pltpu.VMEM((C, 2, PADW), jnp.float32),
        pltpu.SemaphoreType.DMA,
        pltpu.SemaphoreType.DMA,
    ],
    compiler_params=_sc_params,
)
def _pad_kernel(img_hbm, pad_hbm, rbuf, wbuf, sem_in, sem_out):
    wid = lax.axis_index("s") * 2 + lax.axis_index("c")

    r0 = wid * 16
    r2 = wid * 2
    in_copies = []
    for c in range(C):
        in_copies.append(pltpu.async_copy(
            img_hbm.at[c, pl.ds(r0, 16), :],
            rbuf.at[c, :, pl.ds(0, W)], sem_in))
        in_copies.append(pltpu.async_copy(
            img_hbm.at[c, pl.ds(r0, 16), pl.ds(0, PW)],
            rbuf.at[c, :, pl.ds(W, PW)], sem_in))
        in_copies.append(pltpu.async_copy(
            img_hbm.at[c, pl.ds(r2, 2), :],
            wbuf.at[c, :, pl.ds(0, W)], sem_in))
        in_copies.append(pltpu.async_copy(
            img_hbm.at[c, pl.ds(r2, 2), pl.ds(0, PW)],
            wbuf.at[c, :, pl.ds(W, PW)], sem_in))
    for cpy in in_copies:
        cpy.wait()
    out_copies = []
    for c in range(C):
        out_copies.append(pltpu.async_copy(
            rbuf.at[c], pad_hbm.at[c, pl.ds(r0, 16), :], sem_out))
        out_copies.append(pltpu.async_copy(
            wbuf.at[c], pad_hbm.at[c, pl.ds(H + r2, 2), :], sem_out))
    for cpy in out_copies:
        cpy.wait()


@functools.partial(
    pl.kernel,
    out_type=jax.ShapeDtypeStruct((NB, NS * C, PH, PW), jnp.float32),
    mesh=_vector_mesh,
    scratch_types=[
        pltpu.VMEM((NK,), jnp.int32),
        pltpu.VMEM((NK,), jnp.int32),
        pltpu.VMEM((NS * C, PH, WIN), jnp.float32),
        pltpu.VMEM((NS * C, PH, WIN), jnp.float32),
        pltpu.VMEM((NS * C, PH, PW), jnp.float32),
        pltpu.VMEM((NS * C, PH, PW), jnp.float32),
        pltpu.SemaphoreType.DMA,
        pltpu.SemaphoreType.DMA,
    ],
    compiler_params=_sc_params,
)
def _gather_kernel(pad_hbm, ys_hbm, xs_hbm, out_hbm, ys_v, xs_v,
                   win0, win1, opl0, opl1, sem_in, sem_out):
    wid = lax.axis_index("s") * 2 + lax.axis_index("c")

    cy = pltpu.async_copy(ys_hbm, ys_v, sem_in)
    cx = pltpu.async_copy(xs_hbm, xs_v, sem_in)
    cy.wait()
    cx.wait()

    grp = 16 * (wid // 4)
    ys_grp = ys_v[pl.ds(grp, 16)]
    xs_grp = xs_v[pl.ds(grp, 16)]
    lane = lax.iota(jnp.int32, 16)
    lane_base = (wid % 4) * 4

    def coords(u, s):
        t = u * NS + s
        y = jnp.sum(jnp.where(lane == lane_base + t, ys_grp, 0))
        x = jnp.sum(jnp.where(lane == lane_base + t, xs_grp, 0))
        xa = pl.multiple_of(lax.bitwise_and(x, -8), 8)
        return y, xa, x - lax.bitwise_and(x, -8)

    wins = (win0, win1)
    opls = (opl0, opl1)

    def fire_in(u):
        copies = []
        for s in range(NS):
            y, xa, d = coords(u, s)
            for c in range(C):
                copies.append(pltpu.async_copy(
                    pad_hbm.at[c, pl.ds(y, PH), pl.ds(xa, WIN)],
                    wins[u].at[s * C + c], sem_in))
        return copies

    col_const = [16 * v + lane for v in range(PW // 16)]

    in_copies = fire_in(0)
    out_copies = []
    for u in range(2):
        b = wid * 2 + u
        for cpy in in_copies:
            cpy.wait()
        if u == 0:
            in_copies = fire_in(1)

        d_vecs = []
        for s in range(NS):
            _, _, d = coords(u, s)
            d_vecs.append(jnp.full((16,), d, dtype=jnp.int32))

        def body(i, u=u):
            i_vec = jnp.full((16,), i, dtype=jnp.int32)
            for p in range(NS * C):
                p_vec = jnp.full((16,), p, dtype=jnp.int32)
                src_cols = [d_vecs[p // C] + cc for cc in col_const]
                for v in range(PW // 16):
                    x = plsc.load_gather(
                        wins[u], [p_vec, i_vec, src_cols[v]])
                    opls[u][p, i, pl.ds(16 * v, 16)] = x

        pl.loop(0, PH)(body)
        out_copies.append(
            pltpu.async_copy(opls[u], out_hbm.at[b], sem_out))
    for cpy in out_copies:
        cpy.wait()


def kernel(img, dummy, ys, xs):
    del dummy
    imgp = jnp.transpose(img.reshape(H, W, C), (2, 0, 1))
    pad = _pad_kernel(imgp)
    outp = _gather_kernel(pad, ys, xs)
    return jnp.transpose(outp, (0, 2, 3, 1))
